# baseline (speedup 1.0000x reference)
import functools
import struct

import jax
import jax.numpy as jnp
from jax import lax
from jax.experimental import pallas as pl
from jax.experimental.pallas import tpu as pltpu
from jax.experimental.pallas import tpu_sc as plsc

P = 2097152
NC = 2
NS = 16
NW = NC * NS
L = 16
NB = 1024
U_CLIP = 8.0
SCALE = NB / (2.0 * U_CLIP)
PW = P // NW
CHUNK = 16384
NCHUNK = PW // CHUNK
NVREG = CHUNK // L

NBR = 8
NBC = 128
ROWS = 2 * NBR
CWORDS = 2 * NB * L
_NEG_SCALE_BITS = struct.unpack("<i", struct.pack("<f", -SCALE))[0]


def _hist_body(x_hbm, t_hbm, counts_out,
               xb0, tb0, xb1, tb1, counts_v, cred_v,
               sx0, st0, sx1, st1):
    wid = lax.axis_index("s") * NC + lax.axis_index("c")
    zero16 = jnp.zeros((L,), jnp.float32)
    xbufs, tbufs = (xb0, xb1), (tb0, tb1)
    sems = ((sx0, st0), (sx1, st1))

    def start(c):
        p = c % 2
        base = wid * PW + c * CHUNK
        hx = pltpu.async_copy(x_hbm.at[pl.ds(base, CHUNK)], xbufs[p], sems[p][0])
        ht = pltpu.async_copy(t_hbm.at[pl.ds(base, CHUNK)], tbufs[p], sems[p][1])
        return hx, ht

    pending = start(0)

    @plsc.parallel_loop(0, CWORDS // L, 1, unroll=8)
    def _(i):
        counts_v[pl.ds(i * L, L)] = zero16

    lane = lax.iota(jnp.int32, L)
    ones16 = jnp.ones((L,), jnp.float32)

    for c in range(NCHUNK):
        nxt = start(c + 1) if c + 1 < NCHUNK else None
        pending[0].wait()
        pending[1].wait()
        xbuf, tbuf = xbufs[c % 2], tbufs[c % 2]

        @plsc.parallel_loop(0, NVREG, 1, unroll=4)
        def _(j):
            x = xbuf[pl.ds(j * L, L)]
            t = tbuf[pl.ds(j * L, L)]
            s = plsc.bitcast((t << 31) ^ _NEG_SCALE_BITS, jnp.float32)
            kf = x * s + (0.5 * NB)
            kf = jnp.minimum(jnp.maximum(kf, 0.0), float(NB - 1))
            k = kf.astype(jnp.int32)
            cidx = (lane * (2 * NB) + t * NB) | k
            plsc.addupdate_scatter(counts_v, [cidx], ones16)

        pending = nxt

    @plsc.parallel_loop(0, ROWS * (NBC // L), 1, unroll=2)
    def _(i):
        o = i // (NBC // L)
        j = i % (NBC // L)
        acc = counts_v[pl.ds(o * NBC + j * L, L)]
        for ln in range(1, L):
            acc = acc + counts_v[pl.ds(ln * (2 * NB) + o * NBC + j * L, L)]
        cred_v[o, pl.ds(j * L, L)] = acc

    pltpu.sync_copy(cred_v, counts_out.at[wid])


@functools.cache
def _hist_kernel():
    return pl.kernel(
        _hist_body,
        out_type=jax.ShapeDtypeStruct((NW, ROWS, NBC), jnp.float32),
        mesh=plsc.VectorSubcoreMesh(
            core_axis_name="c", subcore_axis_name="s",
            num_cores=NC, num_subcores=NS,
        ),
        scratch_types=(
            pltpu.VMEM((CHUNK,), jnp.float32),
            pltpu.VMEM((CHUNK,), jnp.int32),
            pltpu.VMEM((CHUNK,), jnp.float32),
            pltpu.VMEM((CHUNK,), jnp.int32),
            pltpu.VMEM((CWORDS,), jnp.float32),
            pltpu.VMEM((ROWS, NBC), jnp.float32),
            pltpu.SemaphoreType.DMA,
            pltpu.SemaphoreType.DMA,
            pltpu.SemaphoreType.DMA,
            pltpu.SemaphoreType.DMA,
        ),
        compiler_params=pltpu.CompilerParams(needs_layout_passes=False),
    )


def _finalize_body(c_ref, o_ref):
    csum = jnp.sum(c_ref[...], axis=0)
    m0 = csum[0:NBR, :]
    m1 = csum[NBR:ROWS, :]
    m = m0 + m1
    G = jnp.sum(m1)

    f32 = jnp.float32
    iu = lax.broadcasted_iota(jnp.int32, (NBC, NBC), 0)
    ju = lax.broadcasted_iota(jnp.int32, (NBC, NBC), 1)
    Ut = (iu <= ju).astype(f32)
    il = lax.broadcasted_iota(jnp.int32, (NBR, NBR), 0)
    jl = lax.broadcasted_iota(jnp.int32, (NBR, NBR), 1)
    Ls = (jl < il).astype(f32)

    rowcum_n = jnp.dot(m, Ut, preferred_element_type=f32)
    rowcum_f = jnp.dot(m1, Ut, preferred_element_type=f32)
    prev_n = jnp.dot(Ls, rowcum_n[:, NBC - 1:NBC], preferred_element_type=f32)
    prev_f = jnp.dot(Ls, rowcum_f[:, NBC - 1:NBC], preferred_element_type=f32)
    cum_n = rowcum_n + prev_n
    cum_f = rowcum_f + prev_f

    j_end = 1.0 - (G - cum_f) / (G + cum_n - cum_f)
    e_n = cum_n - m
    e_f = cum_f - m1
    j_start = 1.0 - (G - e_f) / (G + e_n - e_f)

    kr = lax.broadcasted_iota(jnp.int32, (NBR, NBC), 0)
    kc = lax.broadcasted_iota(jnp.int32, (NBR, NBC), 1)
    u_mid = ((kr * NBC + kc).astype(f32) + 0.5) * (1.0 / SCALE) - U_CLIP
    mid_e = 1.0 / (1.0 + jnp.exp(u_mid))
    o_ref[0, 0] = jnp.sum(mid_e * (j_end - j_start))


_finalize_kernel = pl.pallas_call(
    _finalize_body,
    out_shape=jax.ShapeDtypeStruct((1, 1), jnp.float32),
    out_specs=pl.BlockSpec(memory_space=pltpu.SMEM),
)


def kernel(outputs, targets):
    counts_all = _hist_kernel()(outputs, targets)
    loss = _finalize_kernel(counts_all)
    return loss[0, 0]


# speedup vs baseline: 1.0389x; 1.0389x over previous
import functools

import jax
import jax.numpy as jnp
from jax import lax
from jax.experimental import pallas as pl
from jax.experimental.pallas import tpu as pltpu
from jax.experimental.pallas import tpu_sc as plsc

P = 2097152
NC = 2
NS = 16
NW = NC * NS
L = 16
NB = 1024
U_CLIP = 8.0
SCALE = NB / (2.0 * U_CLIP)
PW = P // NW
CHUNK = 16384
NCHUNK = PW // CHUNK
NVREG = CHUNK // L

NBR = 8
NBC = 128
ROWS = 2 * NBR
CWORDS = 2 * NB * L


def _hist_body(x_hbm, t_hbm, counts_out,
               xb0, tb0, xb1, tb1, counts_v, cred_v,
               sx0, st0, sx1, st1):
    wid = lax.axis_index("s") * NC + lax.axis_index("c")
    zero16 = jnp.zeros((L,), jnp.float32)
    xbufs, tbufs = (xb0, xb1), (tb0, tb1)
    sems = ((sx0, st0), (sx1, st1))

    def start(c):
        p = c % 2
        base = wid * PW + c * CHUNK
        hx = pltpu.async_copy(x_hbm.at[pl.ds(base, CHUNK)], xbufs[p], sems[p][0])
        ht = pltpu.async_copy(t_hbm.at[pl.ds(base, CHUNK)], tbufs[p], sems[p][1])
        return hx, ht

    pending = start(0)

    @plsc.parallel_loop(0, CWORDS // L, 1, unroll=8)
    def _(i):
        counts_v[pl.ds(i * L, L)] = zero16

    lane = lax.iota(jnp.int32, L)
    ones16 = jnp.ones((L,), jnp.float32)

    for c in range(NCHUNK):
        nxt = start(c + 1) if c + 1 < NCHUNK else None
        pending[0].wait()
        pending[1].wait()
        xbuf, tbuf = xbufs[c % 2], tbufs[c % 2]

        @plsc.parallel_loop(0, NVREG, 1, unroll=8)
        def _(j):
            x = xbuf[pl.ds(j * L, L)]
            t = tbuf[pl.ds(j * L, L)]
            kf = x * SCALE + (0.5 * NB)
            kf = jnp.minimum(jnp.maximum(kf, 0.0), float(NB - 1))
            k = kf.astype(jnp.int32)
            cidx = (lane * (2 * NB) + t * NB) | k
            plsc.addupdate_scatter(counts_v, [cidx], ones16)

        pending = nxt

    @plsc.parallel_loop(0, ROWS * (NBC // L), 1, unroll=2)
    def _(i):
        o = i // (NBC // L)
        j = i % (NBC // L)
        acc = counts_v[pl.ds(o * NBC + j * L, L)]
        for ln in range(1, L):
            acc = acc + counts_v[pl.ds(ln * (2 * NB) + o * NBC + j * L, L)]
        cred_v[o, pl.ds(j * L, L)] = acc

    pltpu.sync_copy(cred_v, counts_out.at[wid])


@functools.cache
def _hist_kernel():
    return pl.kernel(
        _hist_body,
        out_type=jax.ShapeDtypeStruct((NW, ROWS, NBC), jnp.float32),
        mesh=plsc.VectorSubcoreMesh(
            core_axis_name="c", subcore_axis_name="s",
            num_cores=NC, num_subcores=NS,
        ),
        scratch_types=(
            pltpu.VMEM((CHUNK,), jnp.float32),
            pltpu.VMEM((CHUNK,), jnp.int32),
            pltpu.VMEM((CHUNK,), jnp.float32),
            pltpu.VMEM((CHUNK,), jnp.int32),
            pltpu.VMEM((CWORDS,), jnp.float32),
            pltpu.VMEM((ROWS, NBC), jnp.float32),
            pltpu.SemaphoreType.DMA,
            pltpu.SemaphoreType.DMA,
            pltpu.SemaphoreType.DMA,
            pltpu.SemaphoreType.DMA,
        ),
        compiler_params=pltpu.CompilerParams(needs_layout_passes=False),
    )


def _finalize_body(c_ref, o_ref):
    f32 = jnp.float32
    csum = jnp.sum(c_ref[...], axis=0)
    m0s = csum[0:NBR, :]
    m1 = csum[NBR:ROWS, :]
    ir = lax.broadcasted_iota(jnp.int32, (NBR, NBR), 0)
    jr = lax.broadcasted_iota(jnp.int32, (NBR, NBR), 1)
    Jr = (ir + jr == NBR - 1).astype(f32)
    ic = lax.broadcasted_iota(jnp.int32, (NBC, NBC), 0)
    jc = lax.broadcasted_iota(jnp.int32, (NBC, NBC), 1)
    Jc = (ic + jc == NBC - 1).astype(f32)
    m0 = jnp.dot(Jr, jnp.dot(m0s, Jc, preferred_element_type=f32),
                 preferred_element_type=f32)
    m = m0 + m1
    G = jnp.sum(m1)

    iu = lax.broadcasted_iota(jnp.int32, (NBC, NBC), 0)
    ju = lax.broadcasted_iota(jnp.int32, (NBC, NBC), 1)
    Ut = (iu <= ju).astype(f32)
    il = lax.broadcasted_iota(jnp.int32, (NBR, NBR), 0)
    jl = lax.broadcasted_iota(jnp.int32, (NBR, NBR), 1)
    Ls = (jl < il).astype(f32)

    rowcum_n = jnp.dot(m, Ut, preferred_element_type=f32)
    rowcum_f = jnp.dot(m1, Ut, preferred_element_type=f32)
    prev_n = jnp.dot(Ls, rowcum_n[:, NBC - 1:NBC], preferred_element_type=f32)
    prev_f = jnp.dot(Ls, rowcum_f[:, NBC - 1:NBC], preferred_element_type=f32)
    cum_n = rowcum_n + prev_n
    cum_f = rowcum_f + prev_f

    j_end = 1.0 - (G - cum_f) / (G + cum_n - cum_f)
    e_n = cum_n - m
    e_f = cum_f - m1
    j_start = 1.0 - (G - e_f) / (G + e_n - e_f)

    kr = lax.broadcasted_iota(jnp.int32, (NBR, NBC), 0)
    kc = lax.broadcasted_iota(jnp.int32, (NBR, NBC), 1)
    u_mid = ((kr * NBC + kc).astype(f32) + 0.5) * (1.0 / SCALE) - U_CLIP
    mid_e = 1.0 / (1.0 + jnp.exp(u_mid))
    o_ref[0, 0] = jnp.sum(mid_e * (j_end - j_start))


_finalize_kernel = pl.pallas_call(
    _finalize_body,
    out_shape=jax.ShapeDtypeStruct((1, 1), jnp.float32),
    out_specs=pl.BlockSpec(memory_space=pltpu.SMEM),
)


def kernel(outputs, targets):
    counts_all = _hist_kernel()(outputs, targets)
    loss = _finalize_kernel(counts_all)
    return loss[0, 0]
